# Initial kernel scaffold; baseline (speedup 1.0000x reference)
import jax
import jax.numpy as jnp
from jax.experimental import pallas as pl


def kernel(x, pos, edge_index, batch, sigma, wx, wp):
    raise NotImplementedError("write your pallas kernel here")


# speedup vs baseline: 1.8818x; 1.8818x over previous
import jax
import jax.numpy as jnp
from jax import lax
from jax.experimental import pallas as pl
from jax.experimental.pallas import tpu as pltpu

_SIGMA_DATA = 0.5


def _fused_main_kernel(x_ref, pos_ref, sigma_ref, batch_row_ref,
                       wxm_ref, wpm_ref, wxn_ref, wpn_ref,
                       dx_out_ref, aux_ref, psums_ref):
    @pl.when(pl.program_id(1) == 0)
    def _():
        psums_ref[...] = jnp.zeros_like(psums_ref)

    x = x_ref[...]
    pos = pos_ref[...]
    sigma = sigma_ref[...]
    tm, _ = x.shape
    num_graphs = psums_ref.shape[1]

    sd2 = _SIGMA_DATA * _SIGMA_DATA
    c_in = lax.rsqrt(sd2 + sigma * sigma)
    c_skip = sd2 * c_in * c_in
    c_out = sigma * _SIGMA_DATA * c_in
    c_noise = jnp.log(sigma) * 0.25
    a = c_skip + c_out * c_in

    cx = (c_in * x).astype(jnp.bfloat16)
    dx = jnp.tanh(
        jnp.dot(cx, wxm_ref[...], preferred_element_type=jnp.float32)
        + c_noise * wxn_ref[...])
    delta = jnp.tanh(
        jnp.dot(cx, wpm_ref[...], preferred_element_type=jnp.float32)
        + c_noise * wpn_ref[...])

    dx_out_ref[...] = a * x - c_out * dx
    aux_ref[...] = jnp.concatenate(
        [a * pos + c_out * delta, c_out], axis=-1)

    batch_row = batch_row_ref[...]
    gid_bt = lax.broadcasted_iota(jnp.int32, (num_graphs, tm), 0)
    onehot_bt = (batch_row == gid_bt).astype(jnp.bfloat16)
    delta_aug = jnp.concatenate(
        [delta, jnp.ones((tm, 1), jnp.float32)],
        axis=-1).astype(jnp.bfloat16)
    psums_ref[0, ...] += jnp.dot(onehot_bt, delta_aug,
                                 preferred_element_type=jnp.float32)


def _pos_epilogue_kernel(aux_ref, batch_col_ref, psums_ref, dpos_ref):
    aux = aux_ref[...]
    tm2 = aux.shape[0]
    num_graphs = psums_ref.shape[1]

    sums = jnp.sum(psums_ref[...], axis=0)
    counts = jnp.maximum(sums[:, 3:4], 1.0)
    means = sums[:, :3] / counts

    batch_col = batch_col_ref[...]
    gid_tb = lax.broadcasted_iota(jnp.int32, (tm2, num_graphs), 1)
    onehot_tb = (batch_col == gid_tb).astype(jnp.float32)
    means_per_node = jnp.dot(onehot_tb, means,
                             preferred_element_type=jnp.float32)

    dpos_ref[...] = aux[:, :3] - aux[:, 3:4] * means_per_node


def kernel(x, pos, edge_index, batch, sigma, wx, wp):
    del edge_index
    n, f = x.shape
    num_graphs = 256

    sigma2d = sigma.reshape(n, 1).astype(jnp.float32)
    batch_row = batch.reshape(1, n).astype(jnp.int32)
    batch_col = batch.reshape(n, 1).astype(jnp.int32)

    wxm = wx[:f, :].astype(jnp.bfloat16)
    wpm = wp[:f, :].astype(jnp.bfloat16)
    wxn = wx[f:f + 1, :]
    wpn = wp[f:f + 1, :]

    if n % 4096 == 0:
        tm, outer = 2048, 2
    elif n % 16 == 0:
        tm, outer = n // 2, 2
    else:
        tm, outer = n, 1
    inner = n // (tm * outer)

    dx_out, aux, psums = pl.pallas_call(
        _fused_main_kernel,
        out_shape=(
            jax.ShapeDtypeStruct((n, f), jnp.float32),
            jax.ShapeDtypeStruct((n, 4), jnp.float32),
            jax.ShapeDtypeStruct((outer, num_graphs, 4), jnp.float32),
        ),
        grid=(outer, inner),
        in_specs=[
            pl.BlockSpec((tm, f), lambda o, i, k=inner: (o * k + i, 0)),
            pl.BlockSpec((tm, 3), lambda o, i, k=inner: (o * k + i, 0)),
            pl.BlockSpec((tm, 1), lambda o, i, k=inner: (o * k + i, 0)),
            pl.BlockSpec((1, tm), lambda o, i, k=inner: (0, o * k + i)),
            pl.BlockSpec((f, f), lambda o, i: (0, 0)),
            pl.BlockSpec((f, 3), lambda o, i: (0, 0)),
            pl.BlockSpec((1, f), lambda o, i: (0, 0)),
            pl.BlockSpec((1, 3), lambda o, i: (0, 0)),
        ],
        out_specs=(
            pl.BlockSpec((tm, f), lambda o, i, k=inner: (o * k + i, 0)),
            pl.BlockSpec((tm, 4), lambda o, i, k=inner: (o * k + i, 0)),
            pl.BlockSpec((1, num_graphs, 4), lambda o, i: (o, 0, 0)),
        ),
        compiler_params=pltpu.CompilerParams(
            dimension_semantics=("parallel", "arbitrary"),
            vmem_limit_bytes=64 * 1024 * 1024),
    )(x, pos, sigma2d, batch_row, wxm, wpm, wxn, wpn)

    tm2 = 4096 if n % 4096 == 0 else n
    dpos = pl.pallas_call(
        _pos_epilogue_kernel,
        out_shape=jax.ShapeDtypeStruct((n, 3), jnp.float32),
        grid=(n // tm2,),
        in_specs=[
            pl.BlockSpec((tm2, 4), lambda j: (j, 0)),
            pl.BlockSpec((tm2, 1), lambda j: (j, 0)),
            pl.BlockSpec((outer, num_graphs, 4), lambda j: (0, 0, 0)),
        ],
        out_specs=pl.BlockSpec((tm2, 3), lambda j: (j, 0)),
        compiler_params=pltpu.CompilerParams(
            dimension_semantics=("parallel",),
            vmem_limit_bytes=64 * 1024 * 1024),
    )(aux, batch_col, psums)

    return dx_out, dpos


# speedup vs baseline: 2.0112x; 1.0688x over previous
import jax
import jax.numpy as jnp
from jax import lax
from jax.experimental import pallas as pl
from jax.experimental.pallas import tpu as pltpu

_SIGMA_DATA = 0.5


def _fused_main_kernel(x_ref, pos_ref, sigma_ref, batch_row_ref,
                       wxm_ref, wpm_ref, wxn_ref, wpn_ref,
                       dx_out_ref, aux_ref, psums_ref):
    @pl.when(pl.program_id(1) == 0)
    def _():
        psums_ref[...] = jnp.zeros_like(psums_ref)

    x = x_ref[...]
    pos = pos_ref[...]
    sigma = sigma_ref[...]
    tm, _ = x.shape
    num_graphs = psums_ref.shape[1]

    sd2 = _SIGMA_DATA * _SIGMA_DATA
    c_in = lax.rsqrt(sd2 + sigma * sigma)
    c_skip = sd2 * c_in * c_in
    c_out = sigma * _SIGMA_DATA * c_in
    c_noise = jnp.log(sigma) * 0.25
    a = c_skip + c_out * c_in

    cx = (c_in * x).astype(jnp.bfloat16)
    dx = jnp.tanh(
        jnp.dot(cx, wxm_ref[...], preferred_element_type=jnp.float32)
        + c_noise * wxn_ref[...])
    delta = jnp.tanh(
        jnp.dot(cx, wpm_ref[...], preferred_element_type=jnp.float32)
        + c_noise * wpn_ref[...])

    dx_out_ref[...] = a * x - c_out * dx
    aux_ref[...] = jnp.concatenate(
        [a * pos + c_out * delta, c_out], axis=-1)

    batch_row = batch_row_ref[...]
    gid_bt = lax.broadcasted_iota(jnp.int32, (num_graphs, tm), 0)
    onehot_bt = (batch_row == gid_bt).astype(jnp.bfloat16)
    delta_aug = jnp.concatenate(
        [delta, jnp.ones((tm, 1), jnp.float32)],
        axis=-1).astype(jnp.bfloat16)
    psums_ref[0, ...] += jnp.dot(onehot_bt, delta_aug,
                                 preferred_element_type=jnp.float32)


def _pos_epilogue_kernel(aux_ref, batch_col_ref, psums_ref, dpos_ref):
    aux = aux_ref[...]
    tm2 = aux.shape[0]
    num_graphs = psums_ref.shape[1]

    sums = jnp.sum(psums_ref[...], axis=0)
    counts = jnp.maximum(sums[:, 3:4], 1.0)
    means = sums[:, :3] / counts

    batch_col = batch_col_ref[...]
    gid_tb = lax.broadcasted_iota(jnp.int32, (tm2, num_graphs), 1)
    onehot_tb = (batch_col == gid_tb).astype(jnp.float32)
    means_per_node = jnp.dot(onehot_tb, means,
                             preferred_element_type=jnp.float32)

    dpos_ref[...] = aux[:, :3] - aux[:, 3:4] * means_per_node


def kernel(x, pos, edge_index, batch, sigma, wx, wp):
    del edge_index
    n, f = x.shape
    num_graphs = 256

    sigma2d = sigma.reshape(n, 1).astype(jnp.float32)
    batch_row = batch.reshape(1, n).astype(jnp.int32)
    batch_col = batch.reshape(n, 1).astype(jnp.int32)

    wxm = wx[:f, :].astype(jnp.bfloat16)
    wpm = wp[:f, :].astype(jnp.bfloat16)
    wxn = wx[f:f + 1, :]
    wpn = wp[f:f + 1, :]

    if n % 8192 == 0:
        tm, outer = 4096, 2
    elif n % 16 == 0:
        tm, outer = n // 2, 2
    else:
        tm, outer = n, 1
    inner = n // (tm * outer)

    dx_out, aux, psums = pl.pallas_call(
        _fused_main_kernel,
        out_shape=(
            jax.ShapeDtypeStruct((n, f), jnp.float32),
            jax.ShapeDtypeStruct((n, 4), jnp.float32),
            jax.ShapeDtypeStruct((outer, num_graphs, 4), jnp.float32),
        ),
        grid=(outer, inner),
        in_specs=[
            pl.BlockSpec((tm, f), lambda o, i, k=inner: (o * k + i, 0)),
            pl.BlockSpec((tm, 3), lambda o, i, k=inner: (o * k + i, 0)),
            pl.BlockSpec((tm, 1), lambda o, i, k=inner: (o * k + i, 0)),
            pl.BlockSpec((1, tm), lambda o, i, k=inner: (0, o * k + i)),
            pl.BlockSpec((f, f), lambda o, i: (0, 0)),
            pl.BlockSpec((f, 3), lambda o, i: (0, 0)),
            pl.BlockSpec((1, f), lambda o, i: (0, 0)),
            pl.BlockSpec((1, 3), lambda o, i: (0, 0)),
        ],
        out_specs=(
            pl.BlockSpec((tm, f), lambda o, i, k=inner: (o * k + i, 0)),
            pl.BlockSpec((tm, 4), lambda o, i, k=inner: (o * k + i, 0)),
            pl.BlockSpec((1, num_graphs, 4), lambda o, i: (o, 0, 0)),
        ),
        compiler_params=pltpu.CompilerParams(
            dimension_semantics=("parallel", "arbitrary"),
            vmem_limit_bytes=64 * 1024 * 1024),
    )(x, pos, sigma2d, batch_row, wxm, wpm, wxn, wpn)

    tm2 = 8192 if n % 8192 == 0 else n
    dpos = pl.pallas_call(
        _pos_epilogue_kernel,
        out_shape=jax.ShapeDtypeStruct((n, 3), jnp.float32),
        grid=(n // tm2,),
        in_specs=[
            pl.BlockSpec((tm2, 4), lambda j: (j, 0)),
            pl.BlockSpec((tm2, 1), lambda j: (j, 0)),
            pl.BlockSpec((outer, num_graphs, 4), lambda j: (0, 0, 0)),
        ],
        out_specs=pl.BlockSpec((tm2, 3), lambda j: (j, 0)),
        compiler_params=pltpu.CompilerParams(
            dimension_semantics=("parallel",),
            vmem_limit_bytes=64 * 1024 * 1024),
    )(aux, batch_col, psums)

    return dx_out, dpos


# speedup vs baseline: 2.6490x; 1.3171x over previous
import jax
import jax.numpy as jnp
from jax import lax
from jax.experimental import pallas as pl
from jax.experimental.pallas import tpu as pltpu

_SIGMA_DATA = 0.5


def _fused_main_kernel(x_ref, pos_ref, sigd_ref, batch_row_ref,
                       wxm_ref, wpm_ref, wxn_ref, wpn_ref,
                       dx_out_ref, auxt_ref, psums_ref):
    @pl.when(pl.program_id(1) == 0)
    def _():
        psums_ref[...] = jnp.zeros_like(psums_ref)

    x = x_ref[...]
    pos = pos_ref[...]
    tm, _ = x.shape
    num_graphs = psums_ref.shape[1]

    sigma = sigd_ref[...].T
    sd2 = _SIGMA_DATA * _SIGMA_DATA
    c_in = lax.rsqrt(sd2 + sigma * sigma)
    c_skip = sd2 * c_in * c_in
    c_out = sigma * _SIGMA_DATA * c_in
    c_noise = jnp.log(sigma) * 0.25
    a = c_skip + c_out * c_in

    cx = (c_in * x).astype(jnp.bfloat16)
    dx = jnp.tanh(
        jnp.dot(cx, wxm_ref[...], preferred_element_type=jnp.float32)
        + c_noise * wxn_ref[...])
    delta = jnp.tanh(
        jnp.dot(cx, wpm_ref[...], preferred_element_type=jnp.float32)
        + c_noise * wpn_ref[...])

    dx_out_ref[...] = a * x - c_out * dx

    aux = jnp.concatenate(
        [a * pos + c_out * delta, c_out], axis=-1)
    auxt_ref[...] = aux.T

    batch_row = batch_row_ref[...]
    gid_bt = lax.broadcasted_iota(jnp.int32, (num_graphs, tm), 0)
    onehot_bt = (batch_row == gid_bt).astype(jnp.bfloat16)
    delta_aug = jnp.concatenate(
        [delta, jnp.ones((tm, 1), jnp.float32)],
        axis=-1).astype(jnp.bfloat16)
    psums_ref[0, ...] += jnp.dot(onehot_bt, delta_aug,
                                 preferred_element_type=jnp.float32)


def _pos_epilogue_kernel(auxt_ref, batch_row_ref, psums_ref, dpos_ref):
    auxt = auxt_ref[...]
    tm2 = auxt.shape[1]
    num_graphs = psums_ref.shape[1]

    sums = jnp.sum(psums_ref[...], axis=0)
    counts = jnp.maximum(sums[:, 3:4], 1.0)
    means_t = (sums[:, :3] / counts).T

    batch_row = batch_row_ref[...]
    gid_bt = lax.broadcasted_iota(jnp.int32, (num_graphs, tm2), 0)
    onehot_bt = (batch_row == gid_bt).astype(jnp.float32)
    mpn_t = jnp.dot(means_t, onehot_bt,
                    preferred_element_type=jnp.float32)

    dpos_t = auxt[:3, :] - auxt[3:4, :] * mpn_t
    dpos_ref[...] = dpos_t.T


def kernel(x, pos, edge_index, batch, sigma, wx, wp):
    del edge_index
    n, f = x.shape
    num_graphs = 256

    sigd = sigma.astype(jnp.float32).reshape(1, n)
    batch_row = batch.reshape(1, n).astype(jnp.int32)

    wxm = wx[:f, :].astype(jnp.bfloat16)
    wpm = wp[:f, :].astype(jnp.bfloat16)
    wxn = wx[f:f + 1, :]
    wpn = wp[f:f + 1, :]

    if n % 8192 == 0:
        tm, outer = 4096, 2
    elif n % 16 == 0:
        tm, outer = n // 2, 2
    else:
        tm, outer = n, 1
    inner = n // (tm * outer)

    dx_out, auxt, psums = pl.pallas_call(
        _fused_main_kernel,
        out_shape=(
            jax.ShapeDtypeStruct((n, f), jnp.float32),
            jax.ShapeDtypeStruct((4, n), jnp.float32),
            jax.ShapeDtypeStruct((outer, num_graphs, 4), jnp.float32),
        ),
        grid=(outer, inner),
        in_specs=[
            pl.BlockSpec((tm, f), lambda o, i, k=inner: (o * k + i, 0)),
            pl.BlockSpec((tm, 3), lambda o, i, k=inner: (o * k + i, 0)),
            pl.BlockSpec((1, tm), lambda o, i, k=inner: (0, o * k + i)),
            pl.BlockSpec((1, tm), lambda o, i, k=inner: (0, o * k + i)),
            pl.BlockSpec((f, f), lambda o, i: (0, 0)),
            pl.BlockSpec((f, 3), lambda o, i: (0, 0)),
            pl.BlockSpec((1, f), lambda o, i: (0, 0)),
            pl.BlockSpec((1, 3), lambda o, i: (0, 0)),
        ],
        out_specs=(
            pl.BlockSpec((tm, f), lambda o, i, k=inner: (o * k + i, 0)),
            pl.BlockSpec((4, tm), lambda o, i, k=inner: (0, o * k + i)),
            pl.BlockSpec((1, num_graphs, 4), lambda o, i: (o, 0, 0)),
        ),
        compiler_params=pltpu.CompilerParams(
            dimension_semantics=("parallel", "arbitrary"),
            vmem_limit_bytes=64 * 1024 * 1024),
    )(x, pos, sigd, batch_row, wxm, wpm, wxn, wpn)

    tm2 = 8192 if n % 8192 == 0 else n
    dpos = pl.pallas_call(
        _pos_epilogue_kernel,
        out_shape=jax.ShapeDtypeStruct((n, 3), jnp.float32),
        grid=(n // tm2,),
        in_specs=[
            pl.BlockSpec((4, tm2), lambda j: (0, j)),
            pl.BlockSpec((1, tm2), lambda j: (0, j)),
            pl.BlockSpec((outer, num_graphs, 4), lambda j: (0, 0, 0)),
        ],
        out_specs=pl.BlockSpec((tm2, 3), lambda j: (j, 0)),
        compiler_params=pltpu.CompilerParams(
            dimension_semantics=("parallel",),
            vmem_limit_bytes=64 * 1024 * 1024),
    )(auxt, batch_row, psums)

    return dx_out, dpos


# speedup vs baseline: 2.9274x; 1.1051x over previous
import jax
import jax.numpy as jnp
from jax import lax
from jax.experimental import pallas as pl
from jax.experimental.pallas import tpu as pltpu

_SIGMA_DATA = 0.5


def _fused_main_kernel(x_ref, pos_ref, sigd_ref, batch_row_ref,
                       wxm_ref, wpm_ref, wxn_ref, wpnt_ref,
                       dx_out_ref, auxt_ref, psums_ref):
    @pl.when(pl.program_id(1) == 0)
    def _():
        psums_ref[...] = jnp.zeros_like(psums_ref)

    x = x_ref[...]
    tm, _ = x.shape
    num_graphs = psums_ref.shape[1]
    step = pl.program_id(1)

    sig_r = sigd_ref[pl.ds(step * tm, tm)].reshape(1, tm)
    sd2 = _SIGMA_DATA * _SIGMA_DATA
    ci_r = lax.rsqrt(sd2 + sig_r * sig_r)
    co_r = sig_r * _SIGMA_DATA * ci_r
    cn_r = jnp.log(sig_r) * 0.25
    a_r = sd2 * ci_r * ci_r + co_r * ci_r
    coef = jnp.concatenate([ci_r, cn_r, co_r, a_r], axis=0).T
    c_in = coef[:, 0:1]
    c_noise = coef[:, 1:2]
    c_out = coef[:, 2:3]
    a = coef[:, 3:4]

    cx = (c_in * x).astype(jnp.bfloat16)
    dx = jnp.tanh(
        jnp.dot(cx, wxm_ref[...], preferred_element_type=jnp.float32)
        + c_noise * wxn_ref[...])
    dx_out_ref[...] = a * x - c_out * dx

    dtp = lax.dot_general(wpm_ref[...], cx, (((0,), (1,)), ((), ())),
                          preferred_element_type=jnp.float32)
    delta_t = jnp.tanh(dtp + wpnt_ref[...] * cn_r)
    pos_t = pos_ref[...].T
    auxt_ref[...] = jnp.concatenate(
        [a_r * pos_t + co_r * delta_t, co_r], axis=0)

    batch_row = batch_row_ref[pl.ds(step * tm, tm)].reshape(1, tm)
    gid_bt = lax.broadcasted_iota(jnp.int32, (num_graphs, tm), 0)
    onehot_bt = (batch_row == gid_bt).astype(jnp.bfloat16)
    delta_aug_t = jnp.concatenate(
        [delta_t, jnp.ones((1, tm), jnp.float32)],
        axis=0).astype(jnp.bfloat16)
    psums_ref[0, ...] += lax.dot_general(
        onehot_bt, delta_aug_t, (((1,), (1,)), ((), ())),
        preferred_element_type=jnp.float32)


def _pos_epilogue_kernel(auxt_ref, batch_row_ref, psums_ref, dpos_ref):
    auxt = auxt_ref[...]
    tm2 = auxt.shape[1]
    num_graphs = psums_ref.shape[1]

    sums = jnp.sum(psums_ref[...], axis=0)
    counts = jnp.maximum(sums[:, 3:4], 1.0)
    means_t = (sums[:, :3] / counts).T

    step = pl.program_id(0)
    batch_row = batch_row_ref[pl.ds(step * tm2, tm2)].reshape(1, tm2)
    gid_bt = lax.broadcasted_iota(jnp.int32, (num_graphs, tm2), 0)
    onehot_bt = (batch_row == gid_bt).astype(jnp.float32)
    mpn_t = jnp.dot(means_t, onehot_bt,
                    preferred_element_type=jnp.float32)

    dpos_t = auxt[:3, :] - auxt[3:4, :] * mpn_t
    dpos_ref[...] = dpos_t.T


def kernel(x, pos, edge_index, batch, sigma, wx, wp):
    del edge_index
    n, f = x.shape
    num_graphs = 256

    sigd = sigma.astype(jnp.float32)
    batch_row = batch.astype(jnp.int32)

    wxm = wx[:f, :].astype(jnp.bfloat16)
    wpm = wp[:f, :].astype(jnp.bfloat16)
    wxn = wx[f:f + 1, :]
    wpnt = wp[f:f + 1, :].T

    if n % 8192 == 0:
        tm, outer = 4096, 2
    elif n % 16 == 0:
        tm, outer = n // 2, 2
    else:
        tm, outer = n, 1
    inner = n // (tm * outer)

    dx_out, auxt, psums = pl.pallas_call(
        _fused_main_kernel,
        out_shape=(
            jax.ShapeDtypeStruct((n, f), jnp.float32),
            jax.ShapeDtypeStruct((4, n), jnp.float32),
            jax.ShapeDtypeStruct((outer, num_graphs, 4), jnp.float32),
        ),
        grid=(outer, inner),
        in_specs=[
            pl.BlockSpec((tm, f), lambda o, i, k=inner: (o * k + i, 0)),
            pl.BlockSpec((tm, 3), lambda o, i, k=inner: (o * k + i, 0)),
            pl.BlockSpec((n // outer,), lambda o, i: (o,)),
            pl.BlockSpec((n // outer,), lambda o, i: (o,)),
            pl.BlockSpec((f, f), lambda o, i: (0, 0)),
            pl.BlockSpec((f, 3), lambda o, i: (0, 0)),
            pl.BlockSpec((1, f), lambda o, i: (0, 0)),
            pl.BlockSpec((3, 1), lambda o, i: (0, 0)),
        ],
        out_specs=(
            pl.BlockSpec((tm, f), lambda o, i, k=inner: (o * k + i, 0)),
            pl.BlockSpec((4, tm), lambda o, i, k=inner: (0, o * k + i)),
            pl.BlockSpec((1, num_graphs, 4), lambda o, i: (o, 0, 0)),
        ),
        compiler_params=pltpu.CompilerParams(
            dimension_semantics=("parallel", "arbitrary"),
            vmem_limit_bytes=64 * 1024 * 1024),
    )(x, pos, sigd, batch_row, wxm, wpm, wxn, wpnt)

    tm2 = 8192 if n % 8192 == 0 else n
    dpos = pl.pallas_call(
        _pos_epilogue_kernel,
        out_shape=jax.ShapeDtypeStruct((n, 3), jnp.float32),
        grid=(n // tm2,),
        in_specs=[
            pl.BlockSpec((4, tm2), lambda j: (0, j)),
            pl.BlockSpec((n,), lambda j: (0,)),
            pl.BlockSpec((outer, num_graphs, 4), lambda j: (0, 0, 0)),
        ],
        out_specs=pl.BlockSpec((tm2, 3), lambda j: (j, 0)),
        compiler_params=pltpu.CompilerParams(
            dimension_semantics=("parallel",),
            vmem_limit_bytes=64 * 1024 * 1024),
    )(auxt, batch_row, psums)

    return dx_out, dpos


# speedup vs baseline: 2.9518x; 1.0083x over previous
import jax
import jax.numpy as jnp
from jax import lax
from jax.experimental import pallas as pl
from jax.experimental.pallas import tpu as pltpu

_SIGMA_DATA = 0.5


def _fused_main_kernel(x_ref, pos_ref, sigd_ref, batch_row_ref,
                       wm_ref, wn_ref,
                       dx_out_ref, auxt_ref, psums_ref):
    step = pl.program_id(0)

    @pl.when(step == 0)
    def _():
        psums_ref[...] = jnp.zeros_like(psums_ref)

    x = x_ref[...]
    tm, f = x.shape
    num_graphs = psums_ref.shape[0]

    xb = x.astype(jnp.bfloat16)
    mmx = jnp.dot(xb, wm_ref[:, :f], preferred_element_type=jnp.float32)
    dtp = lax.dot_general(wm_ref[:, f:], xb, (((0,), (1,)), ((), ())),
                          preferred_element_type=jnp.float32)

    sig_r = sigd_ref[pl.ds(step * tm, tm)].reshape(1, tm)
    sd2 = _SIGMA_DATA * _SIGMA_DATA
    ci_r = lax.rsqrt(sd2 + sig_r * sig_r)
    co_r = sig_r * _SIGMA_DATA * ci_r
    cn_r = jnp.log(sig_r) * 0.25
    a_r = sd2 * ci_r * ci_r + co_r * ci_r
    coef = jnp.concatenate([ci_r, cn_r, co_r, a_r], axis=0).T
    c_in = coef[:, 0:1]
    c_noise = coef[:, 1:2]
    c_out = coef[:, 2:3]
    a = coef[:, 3:4]

    dx = jnp.tanh(c_in * mmx + c_noise * wn_ref[:, :f])
    dx_out_ref[...] = a * x - c_out * dx

    wpn_t = wn_ref[:, f:].T
    delta_t = jnp.tanh(ci_r * dtp + wpn_t * cn_r)
    pos_t = pos_ref[...].T
    auxt_ref[...] = jnp.concatenate(
        [a_r * pos_t + co_r * delta_t, co_r], axis=0)

    batch_row = batch_row_ref[pl.ds(step * tm, tm)].reshape(1, tm)
    gid_bt = lax.broadcasted_iota(jnp.int32, (num_graphs, tm), 0)
    onehot_bt = (batch_row == gid_bt).astype(jnp.bfloat16)
    delta_aug_t = jnp.concatenate(
        [delta_t, jnp.ones((1, tm), jnp.float32)],
        axis=0).astype(jnp.bfloat16)
    psums_ref[...] += lax.dot_general(
        onehot_bt, delta_aug_t, (((1,), (1,)), ((), ())),
        preferred_element_type=jnp.float32)


def _pos_epilogue_kernel(auxt_ref, batch_row_ref, psums_ref, dpos_ref):
    auxt = auxt_ref[...]
    tm2 = auxt.shape[1]
    num_graphs = psums_ref.shape[0]

    sums = psums_ref[...]
    counts = jnp.maximum(sums[:, 3:4], 1.0)
    means_t = (sums[:, :3] / counts).T

    step = pl.program_id(0)
    batch_row = batch_row_ref[pl.ds(step * tm2, tm2)].reshape(1, tm2)
    gid_bt = lax.broadcasted_iota(jnp.int32, (num_graphs, tm2), 0)
    onehot_bt = (batch_row == gid_bt).astype(jnp.float32)
    mpn_t = jnp.dot(means_t, onehot_bt,
                    preferred_element_type=jnp.float32)

    dpos_t = auxt[:3, :] - auxt[3:4, :] * mpn_t
    dpos_ref[...] = dpos_t.T


def kernel(x, pos, edge_index, batch, sigma, wx, wp):
    del edge_index
    n, f = x.shape
    num_graphs = 256

    sigd = sigma.astype(jnp.float32)
    batch_row = batch.astype(jnp.int32)

    w_main = jnp.concatenate(
        [wx[:f, :], wp[:f, :]], axis=-1).astype(jnp.bfloat16)
    w_noise = jnp.concatenate(
        [wx[f:f + 1, :], wp[f:f + 1, :]], axis=-1)

    if n % 8192 == 0:
        tm = 8192
    elif n % 16 == 0:
        tm = n // 2
    else:
        tm = n
    num_tiles = n // tm

    dx_out, auxt, psums = pl.pallas_call(
        _fused_main_kernel,
        out_shape=(
            jax.ShapeDtypeStruct((n, f), jnp.float32),
            jax.ShapeDtypeStruct((4, n), jnp.float32),
            jax.ShapeDtypeStruct((num_graphs, 4), jnp.float32),
        ),
        grid=(num_tiles,),
        in_specs=[
            pl.BlockSpec((tm, f), lambda i: (i, 0)),
            pl.BlockSpec((tm, 3), lambda i: (i, 0)),
            pl.BlockSpec((n,), lambda i: (0,)),
            pl.BlockSpec((n,), lambda i: (0,)),
            pl.BlockSpec((f, f + 3), lambda i: (0, 0)),
            pl.BlockSpec((1, f + 3), lambda i: (0, 0)),
        ],
        out_specs=(
            pl.BlockSpec((tm, f), lambda i: (i, 0)),
            pl.BlockSpec((4, tm), lambda i: (0, i)),
            pl.BlockSpec((num_graphs, 4), lambda i: (0, 0)),
        ),
        compiler_params=pltpu.CompilerParams(
            dimension_semantics=("arbitrary",),
            vmem_limit_bytes=64 * 1024 * 1024),
    )(x, pos, sigd, batch_row, w_main, w_noise)

    tm2 = 8192 if n % 8192 == 0 else n
    dpos = pl.pallas_call(
        _pos_epilogue_kernel,
        out_shape=jax.ShapeDtypeStruct((n, 3), jnp.float32),
        grid=(n // tm2,),
        in_specs=[
            pl.BlockSpec((4, tm2), lambda j: (0, j)),
            pl.BlockSpec((n,), lambda j: (0,)),
            pl.BlockSpec((num_graphs, 4), lambda j: (0, 0)),
        ],
        out_specs=pl.BlockSpec((tm2, 3), lambda j: (j, 0)),
        compiler_params=pltpu.CompilerParams(
            dimension_semantics=("arbitrary",),
            vmem_limit_bytes=64 * 1024 * 1024),
    )(auxt, batch_row, psums)

    return dx_out, dpos


# speedup vs baseline: 3.0520x; 1.0339x over previous
import jax
import jax.numpy as jnp
from jax import lax
from jax.experimental import pallas as pl
from jax.experimental.pallas import tpu as pltpu

_SIGMA_DATA = 0.5


def _fused_main_kernel(x_ref, pos_ref, sigd_ref, batch_row_ref,
                       wm_ref, wn_ref,
                       dx_out_ref, auxt_ref, psums_ref):
    step = pl.program_id(0)

    @pl.when(step == 0)
    def _():
        psums_ref[...] = jnp.zeros_like(psums_ref)

    x = x_ref[...]
    tm, f = x.shape
    num_graphs = psums_ref.shape[0]

    xb = x.astype(jnp.bfloat16)
    mmx = jnp.dot(xb, wm_ref[:, :f], preferred_element_type=jnp.float32)
    dtp = lax.dot_general(wm_ref[:, f:], xb, (((0,), (1,)), ((), ())),
                          preferred_element_type=jnp.float32)

    sig_r = sigd_ref[pl.ds(step * tm, tm)].reshape(1, tm)
    sd2 = _SIGMA_DATA * _SIGMA_DATA
    ci_r = lax.rsqrt(sd2 + sig_r * sig_r)
    co_r = sig_r * _SIGMA_DATA * ci_r
    cn_r = jnp.log(sig_r) * 0.25
    a_r = sd2 * ci_r * ci_r + co_r * ci_r
    coef = jnp.concatenate([ci_r, cn_r, co_r, a_r], axis=0).T
    c_in = coef[:, 0:1]
    c_noise = coef[:, 1:2]
    c_out = coef[:, 2:3]
    a = coef[:, 3:4]

    mm_bias = jnp.dot(c_noise.astype(jnp.bfloat16),
                      wn_ref[:, :f].astype(jnp.bfloat16),
                      preferred_element_type=jnp.float32)
    dx = jnp.tanh(c_in * mmx + mm_bias)
    dx_out_ref[...] = a * x - c_out * dx

    wpn_t = wn_ref[:, f:].T
    delta_t = jnp.tanh(ci_r * dtp + wpn_t * cn_r)
    pos_t = pos_ref[...].T
    auxt_ref[...] = jnp.concatenate(
        [a_r * pos_t + co_r * delta_t, co_r], axis=0)

    batch_row = batch_row_ref[pl.ds(step * tm, tm)].reshape(1, tm)
    gid_bt = lax.broadcasted_iota(jnp.int32, (num_graphs, tm), 0)
    onehot_bt = (batch_row == gid_bt).astype(jnp.bfloat16)
    delta_aug_t = jnp.concatenate(
        [delta_t, jnp.ones((1, tm), jnp.float32)],
        axis=0).astype(jnp.bfloat16)
    psums_ref[...] += lax.dot_general(
        onehot_bt, delta_aug_t, (((1,), (1,)), ((), ())),
        preferred_element_type=jnp.float32)


def _pos_epilogue_kernel(auxt_ref, batch_row_ref, psums_ref, dpos_ref):
    auxt = auxt_ref[...]
    tm2 = auxt.shape[1]
    num_graphs = psums_ref.shape[0]

    sums = psums_ref[...]
    counts = jnp.maximum(sums[:, 3:4], 1.0)
    means_t = (sums[:, :3] / counts).T

    step = pl.program_id(0)
    batch_row = batch_row_ref[pl.ds(step * tm2, tm2)].reshape(1, tm2)
    gid_bt = lax.broadcasted_iota(jnp.int32, (num_graphs, tm2), 0)
    onehot_bt = (batch_row == gid_bt).astype(jnp.float32)
    mpn_t = jnp.dot(means_t, onehot_bt,
                    preferred_element_type=jnp.float32)

    dpos_t = auxt[:3, :] - auxt[3:4, :] * mpn_t
    dpos_ref[...] = dpos_t.T


def kernel(x, pos, edge_index, batch, sigma, wx, wp):
    del edge_index
    n, f = x.shape
    num_graphs = 256

    sigd = sigma.astype(jnp.float32)
    batch_row = batch.astype(jnp.int32)

    w_main = jnp.concatenate(
        [wx[:f, :], wp[:f, :]], axis=-1).astype(jnp.bfloat16)
    w_noise = jnp.concatenate(
        [wx[f:f + 1, :], wp[f:f + 1, :]], axis=-1)

    if n % 8192 == 0:
        tm = 8192
    elif n % 16 == 0:
        tm = n // 2
    else:
        tm = n
    num_tiles = n // tm

    dx_out, auxt, psums = pl.pallas_call(
        _fused_main_kernel,
        out_shape=(
            jax.ShapeDtypeStruct((n, f), jnp.float32),
            jax.ShapeDtypeStruct((4, n), jnp.float32),
            jax.ShapeDtypeStruct((num_graphs, 4), jnp.float32),
        ),
        grid=(num_tiles,),
        in_specs=[
            pl.BlockSpec((tm, f), lambda i: (i, 0)),
            pl.BlockSpec((tm, 3), lambda i: (i, 0)),
            pl.BlockSpec((n,), lambda i: (0,)),
            pl.BlockSpec((n,), lambda i: (0,)),
            pl.BlockSpec((f, f + 3), lambda i: (0, 0)),
            pl.BlockSpec((1, f + 3), lambda i: (0, 0)),
        ],
        out_specs=(
            pl.BlockSpec((tm, f), lambda i: (i, 0)),
            pl.BlockSpec((4, tm), lambda i: (0, i)),
            pl.BlockSpec((num_graphs, 4), lambda i: (0, 0)),
        ),
        compiler_params=pltpu.CompilerParams(
            dimension_semantics=("arbitrary",),
            vmem_limit_bytes=64 * 1024 * 1024),
    )(x, pos, sigd, batch_row, w_main, w_noise)

    tm2 = 8192 if n % 8192 == 0 else n
    dpos = pl.pallas_call(
        _pos_epilogue_kernel,
        out_shape=jax.ShapeDtypeStruct((n, 3), jnp.float32),
        grid=(n // tm2,),
        in_specs=[
            pl.BlockSpec((4, tm2), lambda j: (0, j)),
            pl.BlockSpec((n,), lambda j: (0,)),
            pl.BlockSpec((num_graphs, 4), lambda j: (0, 0)),
        ],
        out_specs=pl.BlockSpec((tm2, 3), lambda j: (j, 0)),
        compiler_params=pltpu.CompilerParams(
            dimension_semantics=("arbitrary",),
            vmem_limit_bytes=64 * 1024 * 1024),
    )(auxt, batch_row, psums)

    return dx_out, dpos
